# baseline (speedup 1.0000x reference)
import functools
import math

import jax
import jax.numpy as jnp
from jax import lax
from jax.experimental import pallas as pl
from jax.experimental.pallas import tpu as pltpu

_EPS = 1e-5


def _round_up(v, m):
    return ((v + m - 1) // m) * m


def _row_folded_weights(w):
    m = jnp.array([[[1, 0, 0], [0, 1, 1]],
                   [[1, 1, 0], [0, 0, 1]]], dtype=w.dtype)
    ph = jnp.einsum('pak,oikl->paloi', m, w)
    return ph.reshape(12, w.shape[0], w.shape[1])


def _load_planes(x_ref, row0, *, TR, W2, Cin):
    return [x_ref[pl.ds(row0, TR + 2), pl.ds(kw, W2), :].reshape(
        (TR + 2) * W2, Cin) for kw in (0, 1, 2)]


def _row_conv(planes, w_ref, pi, *, TR, W2, Cin, Cout):
    acc = jnp.zeros((Cout, TR * W2), jnp.float32)
    for a in (0, 1):
        for kw in (0, 1, 2):
            sl = planes[kw][(pi + a) * W2:(pi + a + TR) * W2, :]
            acc = acc + lax.dot_general(
                w_ref[pi * 6 + a * 3 + kw], sl,
                dimension_numbers=(((1,), (1,)), ((), ())),
                preferred_element_type=jnp.float32)
    return acc


def _stats_body(x_ref, w_ref, s_ref, q_ref, z_ref, *,
                TR, W2, Cin, Cout, H, W, masked):
    row0 = pl.program_id(1) * TR
    if masked:
        lane = lax.broadcasted_iota(jnp.int32, (1, TR * W2), 1)
        ok = ((row0 + lane // W2) < H) & ((lane % W2) < 2 * W)
    s = jnp.zeros((Cout, 1), jnp.float32)
    q = jnp.zeros((Cout, 1), jnp.float32)
    planes = _load_planes(x_ref, row0, TR=TR, W2=W2, Cin=Cin)
    zs = []
    for pi in (0, 1):
        z = _row_conv(planes, w_ref, pi, TR=TR, W2=W2, Cin=Cin, Cout=Cout)
        if masked:
            z = jnp.where(ok, z, 0.0)
        s = s + jnp.sum(z, axis=1, keepdims=True)
        q = q + jnp.sum(z * z, axis=1, keepdims=True)
        zs.append(z.astype(jnp.bfloat16).reshape(Cout, TR, W2))
    z_ref[...] = jnp.concatenate(zs, axis=2)
    s_ref[...] = s
    q_ref[...] = q


def _affine_body(z_ref, sc_ref, sh_ref, o_ref, *, TR, W2, Cout):
    sc = sc_ref[...].reshape(Cout, 1, 1, 1)
    sh = sh_ref[...].reshape(Cout, 1, 1, 1)
    zv = z_ref[...]
    pair = jnp.stack([zv[:, :, :W2], zv[:, :, W2:]], axis=2)
    o_ref[...] = jnp.maximum(pair.astype(jnp.float32) * sc + sh, 0.0)


@jax.jit
def _forward(x, w, b, gamma, beta):
    N, Cin, H, W = x.shape
    Cout = w.shape[0]

    W2 = _round_up(2 * W, 8)
    g = 128 // math.gcd(W2, 128)
    TR = g * max(1, 4096 // (g * W2))
    if TR >= H:
        TR = H
    Hp = _round_up(H, TR)
    nT = Hp // TR
    masked = (Hp != H) or (W2 != 2 * W)

    xh = jnp.transpose(x, (0, 2, 3, 1)).astype(jnp.bfloat16)
    xe = jnp.repeat(xh, 2, axis=2)
    xe = jnp.pad(xe, ((0, 0), (1, Hp - H + 1), (1, W2 - 2 * W + 1), (0, 0)))
    wph = _row_folded_weights(w.astype(jnp.float32)).astype(jnp.bfloat16)

    x_spec = pl.BlockSpec((None, Hp + 2, W2 + 2, Cin), lambda n, t: (n, 0, 0, 0))
    w_spec = pl.BlockSpec((12, Cout, Cin), lambda n, t: (0, 0, 0))
    p_spec = pl.BlockSpec((None, None, Cout, 1), lambda n, t: (n, t, 0, 0))

    ps, pq, zact = pl.pallas_call(
        functools.partial(_stats_body, TR=TR, W2=W2, Cin=Cin, Cout=Cout,
                          H=H, W=W, masked=masked),
        grid=(N, nT),
        in_specs=[x_spec, w_spec],
        out_specs=(p_spec, p_spec,
                   pl.BlockSpec((None, Cout, TR, 2 * W2),
                                lambda n, t: (n, 0, t, 0))),
        out_shape=(jax.ShapeDtypeStruct((N, nT, Cout, 1), jnp.float32),
                   jax.ShapeDtypeStruct((N, nT, Cout, 1), jnp.float32),
                   jax.ShapeDtypeStruct((N, Cout, Hp, 2 * W2), jnp.bfloat16)),
        compiler_params=pltpu.CompilerParams(
            dimension_semantics=("parallel", "parallel")),
    )(xe, wph)

    count = jnp.float32(N * 4 * H * W)
    mean = jnp.sum(ps, axis=(0, 1)) / count
    var = jnp.maximum(jnp.sum(pq, axis=(0, 1)) / count - mean * mean, 0.0)
    scale = gamma.astype(jnp.float32).reshape(Cout, 1) * lax.rsqrt(var + _EPS)
    shift = beta.astype(jnp.float32).reshape(Cout, 1) - mean * scale

    v_spec = pl.BlockSpec((Cout, 1), lambda n, t: (0, 0))
    zi_spec = pl.BlockSpec((None, Cout, TR, 2 * W2), lambda n, t: (n, 0, t, 0))
    o_spec = pl.BlockSpec((None, Cout, TR, 2, W2), lambda n, t: (n, 0, t, 0, 0))
    out = pl.pallas_call(
        functools.partial(_affine_body, TR=TR, W2=W2, Cout=Cout),
        grid=(N, nT),
        in_specs=[zi_spec, v_spec, v_spec],
        out_specs=o_spec,
        out_shape=jax.ShapeDtypeStruct((N, Cout, Hp, 2, W2), jnp.float32),
        compiler_params=pltpu.CompilerParams(
            dimension_semantics=("parallel", "parallel")),
    )(zact, scale, shift)

    out = out.reshape(N, Cout, 2 * Hp, W2)
    if masked:
        out = out[:, :, :2 * H, :2 * W]
    return out


def kernel(x, w, b, gamma, beta):
    return _forward(x, w, b, gamma, beta)


# speedup vs baseline: 1.4318x; 1.4318x over previous
import functools
import math

import jax
import jax.numpy as jnp
from jax import lax
from jax.experimental import pallas as pl
from jax.experimental.pallas import tpu as pltpu

_EPS = 1e-5


def _round_up(v, m):
    return ((v + m - 1) // m) * m


def _row_folded_weights(w):
    m = jnp.array([[[1, 0, 0], [0, 1, 1]],
                   [[1, 1, 0], [0, 0, 1]]], dtype=w.dtype)
    ph = jnp.einsum('pak,oikl->paloi', m, w)
    return ph.reshape(12, w.shape[0], w.shape[1])


def _load_planes(x_ref, row0, *, TR, W2, Cin):
    return [x_ref[pl.ds(row0, TR + 2), pl.ds(kw, W2), :].reshape(
        (TR + 2) * W2, Cin) for kw in (0, 1, 2)]


def _row_conv(planes, w_ref, pi, *, TR, W2, Cin, Cout):
    acc = jnp.zeros((Cout, TR * W2), jnp.float32)
    for a in (0, 1):
        for kw in (0, 1, 2):
            sl = planes[kw][(pi + a) * W2:(pi + a + TR) * W2, :]
            acc = acc + lax.dot_general(
                w_ref[pi * 6 + a * 3 + kw], sl,
                dimension_numbers=(((1,), (1,)), ((), ())),
                preferred_element_type=jnp.float32)
    return acc


def _stats_body(x_ref, w_ref, s_ref, q_ref, z_ref, *,
                TR, W2, Cin, Cout, H, W, masked):
    row0 = pl.program_id(1) * TR
    if masked:
        lane = lax.broadcasted_iota(jnp.int32, (1, TR * W2), 1)
        ok = ((row0 + lane // W2) < H) & ((lane % W2) < 2 * W)
    s = jnp.zeros((Cout, 1), jnp.float32)
    q = jnp.zeros((Cout, 1), jnp.float32)
    planes = _load_planes(x_ref, row0, TR=TR, W2=W2, Cin=Cin)
    zs = []
    for pi in (0, 1):
        z = _row_conv(planes, w_ref, pi, TR=TR, W2=W2, Cin=Cin, Cout=Cout)
        if masked:
            z = jnp.where(ok, z, 0.0)
        s = s + jnp.sum(z, axis=1, keepdims=True)
        q = q + jnp.sum(z * z, axis=1, keepdims=True)
        zs.append(z.astype(jnp.bfloat16).reshape(Cout, TR, W2))
    z_ref[...] = jnp.concatenate(zs, axis=2)
    s_ref[...] = s
    q_ref[...] = q


def _affine_body(z_ref, sc_ref, sh_ref, o_ref, *, TR, W2, Cout):
    sc = sc_ref[...].reshape(Cout, 1, 1, 1)
    sh = sh_ref[...].reshape(Cout, 1, 1, 1)
    zv = z_ref[...]
    pair = jnp.stack([zv[:, :, :W2], zv[:, :, W2:]], axis=2)
    o_ref[...] = jnp.maximum(pair.astype(jnp.float32) * sc + sh, 0.0)


@jax.jit
def _forward(x, w, b, gamma, beta):
    N, Cin, H, W = x.shape
    Cout = w.shape[0]

    W2 = _round_up(2 * W, 8)
    g = 128 // math.gcd(W2, 128)
    TR = g * max(1, 2048 // (g * W2))
    if TR >= H:
        TR = H
    Hp = _round_up(H, TR)
    nT = Hp // TR
    masked = (Hp != H) or (W2 != 2 * W)

    xh = jnp.transpose(x, (0, 2, 3, 1)).astype(jnp.bfloat16)
    xe = jnp.repeat(xh, 2, axis=2)
    xe = jnp.pad(xe, ((0, 0), (1, Hp - H + 1), (1, W2 - 2 * W + 1), (0, 0)))
    wph = _row_folded_weights(w.astype(jnp.float32)).astype(jnp.bfloat16)

    x_spec = pl.BlockSpec((None, Hp + 2, W2 + 2, Cin), lambda n, t: (n, 0, 0, 0))
    w_spec = pl.BlockSpec((12, Cout, Cin), lambda n, t: (0, 0, 0))
    p_spec = pl.BlockSpec((None, None, Cout, 1), lambda n, t: (n, t, 0, 0))

    ps, pq, zact = pl.pallas_call(
        functools.partial(_stats_body, TR=TR, W2=W2, Cin=Cin, Cout=Cout,
                          H=H, W=W, masked=masked),
        grid=(N, nT),
        in_specs=[x_spec, w_spec],
        out_specs=(p_spec, p_spec,
                   pl.BlockSpec((None, Cout, TR, 2 * W2),
                                lambda n, t: (n, 0, t, 0))),
        out_shape=(jax.ShapeDtypeStruct((N, nT, Cout, 1), jnp.float32),
                   jax.ShapeDtypeStruct((N, nT, Cout, 1), jnp.float32),
                   jax.ShapeDtypeStruct((N, Cout, Hp, 2 * W2), jnp.bfloat16)),
        compiler_params=pltpu.CompilerParams(
            dimension_semantics=("parallel", "parallel")),
    )(xe, wph)

    count = jnp.float32(N * 4 * H * W)
    mean = jnp.sum(ps, axis=(0, 1)) / count
    var = jnp.maximum(jnp.sum(pq, axis=(0, 1)) / count - mean * mean, 0.0)
    scale = gamma.astype(jnp.float32).reshape(Cout, 1) * lax.rsqrt(var + _EPS)
    shift = beta.astype(jnp.float32).reshape(Cout, 1) - mean * scale

    v_spec = pl.BlockSpec((Cout, 1), lambda n, t: (0, 0))
    zi_spec = pl.BlockSpec((None, Cout, TR, 2 * W2), lambda n, t: (n, 0, t, 0))
    o_spec = pl.BlockSpec((None, Cout, TR, 2, W2), lambda n, t: (n, 0, t, 0, 0))
    out = pl.pallas_call(
        functools.partial(_affine_body, TR=TR, W2=W2, Cout=Cout),
        grid=(N, nT),
        in_specs=[zi_spec, v_spec, v_spec],
        out_specs=o_spec,
        out_shape=jax.ShapeDtypeStruct((N, Cout, Hp, 2, W2), jnp.float32),
        compiler_params=pltpu.CompilerParams(
            dimension_semantics=("parallel", "parallel")),
    )(zact, scale, shift)

    out = out.reshape(N, Cout, 2 * Hp, W2)
    if masked:
        out = out[:, :, :2 * H, :2 * W]
    return out


def kernel(x, w, b, gamma, beta):
    return _forward(x, w, b, gamma, beta)
